# baseline (speedup 1.0000x reference)
import functools

import jax
import jax.numpy as jnp
from jax import lax
from jax.experimental import pallas as pl
from jax.experimental.pallas import tpu as pltpu
from jax.experimental.pallas import tpu_sc as plsc

_NUM_CORES = 2
_NUM_SUBCORES = 16
_NW = _NUM_CORES * _NUM_SUBCORES

_B = 4096 * 200
_D = 64
_BPW = _B // _NW
_CHUNK = 512
_NCHUNK = _BPW // _CHUNK


@functools.partial(
    pl.kernel,
    out_type=jax.ShapeDtypeStruct((_B, _D), jnp.float32),
    mesh=plsc.VectorSubcoreMesh(core_axis_name="c", subcore_axis_name="s"),
    scratch_types=[
        pltpu.VMEM((_CHUNK,), jnp.int32),
        pltpu.VMEM((_CHUNK, _D), jnp.float32),
        pltpu.SemaphoreType.DMA,
    ],
    compiler_params=pltpu.CompilerParams(use_tc_tiling_on_sc=False),
)
def _embed_sc(tokens_hbm, table_hbm, out_hbm, idx_v, rows_v, sem):
    wid = lax.axis_index("s") * _NUM_CORES + lax.axis_index("c")
    base = wid * _BPW

    @pl.loop(0, _NCHUNK)
    def _chunk(i):
        off = base + i * _CHUNK
        pltpu.sync_copy(tokens_hbm.at[pl.ds(off, _CHUNK)], idx_v)
        pltpu.async_copy(table_hbm.at[idx_v], rows_v, sem).wait()
        pltpu.sync_copy(rows_v, out_hbm.at[pl.ds(off, _CHUNK)])


def kernel(tokens_id, weight):
    flat = tokens_id.reshape(-1)
    out = _embed_sc(flat, weight)
    return out.reshape(tokens_id.shape + (weight.shape[1],))


# speedup vs baseline: 1.0433x; 1.0433x over previous
import functools

import jax
import jax.numpy as jnp
from jax import lax
from jax.experimental import pallas as pl
from jax.experimental.pallas import tpu as pltpu
from jax.experimental.pallas import tpu_sc as plsc

_NUM_CORES = 2
_NUM_SUBCORES = 16
_NW = _NUM_CORES * _NUM_SUBCORES

_B = 4096 * 200
_D = 64
_BPW = _B // _NW
_CHUNK = 256
_NCHUNK = _BPW // _CHUNK
_NBUF = 4
assert _NCHUNK % _NBUF == 0


@functools.partial(
    pl.kernel,
    out_type=jax.ShapeDtypeStruct((_B, _D), jnp.float32),
    mesh=plsc.VectorSubcoreMesh(core_axis_name="c", subcore_axis_name="s"),
    scratch_types=(
        [
            pltpu.VMEM((_NCHUNK, _CHUNK), jnp.int32),
            pltpu.VMEM((_NBUF, _CHUNK, _D), jnp.float32),
        ]
        + [pltpu.SemaphoreType.DMA] * (2 * _NBUF)
    ),
    compiler_params=pltpu.CompilerParams(use_tc_tiling_on_sc=False),
)
def _embed_sc(tokens_hbm, table_hbm, out_hbm, idx_v, rows_v, *sems):
    gsem = sems[:_NBUF]
    osem = sems[_NBUF:]
    wid = lax.axis_index("s") * _NUM_CORES + lax.axis_index("c")
    base = wid * _BPW

    pltpu.sync_copy(tokens_hbm.at[pl.ds(wid * _NCHUNK, _NCHUNK)], idx_v)

    def start_gather(i, b):
        pltpu.async_copy(table_hbm.at[idx_v.at[i]], rows_v.at[b], gsem[b])

    def wait_gather(b):
        pltpu.make_async_copy(
            out_hbm.at[pl.ds(base, _CHUNK)], rows_v.at[b], gsem[b]
        ).wait()

    def start_out(i, b):
        pltpu.async_copy(
            rows_v.at[b], out_hbm.at[pl.ds(base + i * _CHUNK, _CHUNK)], osem[b]
        )

    def wait_out(b):
        pltpu.make_async_copy(
            rows_v.at[b], out_hbm.at[pl.ds(base, _CHUNK)], osem[b]
        ).wait()

    for b in range(_NBUF):
        start_gather(b, b)

    @pl.loop(0, _NCHUNK, step=_NBUF)
    def _group(g):
        for b in range(_NBUF):
            wait_gather(b)
            start_out(g + b, b)
        for b in range(_NBUF):
            i = g + b

            @pl.when(i + _NBUF < _NCHUNK)
            def _():
                wait_out(b)
                start_gather(i + _NBUF, b)

    for b in range(_NBUF):
        wait_out(b)


def kernel(tokens_id, weight):
    flat = tokens_id.reshape(_B // _CHUNK, _CHUNK)
    out = _embed_sc(flat, weight)
    return out.reshape(tokens_id.shape + (weight.shape[1],))
